# baseline (speedup 1.0000x reference)
import functools

import jax
import jax.numpy as jnp
from jax import lax
from jax.experimental import pallas as pl
from jax.experimental.pallas import tpu as pltpu
from jax.experimental.pallas import tpu_sc as plsc

N_NODES = 10000
N_EDGES = 320000
H = 128
ROW_BLK = 1000

_NC = 2
_NS = 16
_NW = _NC * _NS
_GC = 80


def _sc_mesh():
    return plsc.VectorSubcoreMesh(
        core_axis_name="c", subcore_axis_name="s",
        num_cores=_NC, num_subcores=_NS)


def _sc_gather(table, idx, out_rows=None, row_offset=0):
    b = idx.shape[0]
    per_w = b // _NW
    assert per_w * _NW == b and per_w % _GC == 0
    n_chunks = per_w // _GC

    @functools.partial(
        pl.kernel,
        out_type=jax.ShapeDtypeStruct((out_rows or b, H), jnp.float32),
        mesh=_sc_mesh(),
        scratch_types=[
            pltpu.VMEM((2, _GC), jnp.int32),
            pltpu.VMEM((2, _GC, H), jnp.float32),
            pltpu.SemaphoreType.DMA,
            pltpu.SemaphoreType.DMA,
        ],
    )
    def gather_k(table_hbm, idx_hbm, out_hbm, idx_v, rows_v, sem0, sem1):
        wid = lax.axis_index("s") * _NC + lax.axis_index("c")
        base = wid * per_w
        sems = (sem0, sem1)

        def body(j, p):
            q = 1 - p

            @pl.when(j + 1 < n_chunks)
            def _():
                off = base + (j + 1) * _GC
                pltpu.sync_copy(idx_hbm.at[pl.ds(off, _GC)], idx_v.at[q])
                pltpu.async_copy(table_hbm.at[idx_v.at[q]], rows_v.at[q], sems[q])

            pltpu.make_async_copy(
                table_hbm.at[idx_v.at[p]], rows_v.at[p], sems[p]).wait()
            pltpu.sync_copy(
                rows_v.at[p],
                out_hbm.at[pl.ds(row_offset + base + j * _GC, _GC)])

        pltpu.sync_copy(idx_hbm.at[pl.ds(base, _GC)], idx_v.at[0])
        pltpu.async_copy(table_hbm.at[idx_v.at[0]], rows_v.at[0], sem0)

        @pl.loop(0, 2 * (n_chunks // 2), step=2)
        def _(i):
            body(i, 0)
            body(i + 1, 1)

        if n_chunks % 2:
            body(n_chunks - 1, 0)

    return gather_k(table, idx)


def _zero_vmem_rows(buf, nrows):
    z = jnp.zeros((_L,), jnp.float32)

    @pl.loop(0, nrows)
    def _(i):
        for k in range(H // _L):
            buf[i, pl.ds(k * _L, _L)] = z


def _copy_idx_row(dst2d, src1d, off):
    for k in range(128 // _L):
        dst2d[0, pl.ds(k * _L, _L)] = src1d[pl.ds(off + k * _L, _L)]


def _pad_tail(buf, cnt, value):
    v = jnp.full((_L,), value, jnp.int32) + lax.rem(
        lax.iota(jnp.int32, _L), jnp.int32(8))
    for k in range(128 // _L):
        buf[pl.ds(cnt + k * _L, _L)] = v


def _flush_batches(sbuf, dbuf, nb, sidx, didx, rows, acc, table_hbm, sem):
    @pl.loop(0, nb)
    def _(k):
        off = k * 128
        _copy_idx_row(sidx, sbuf, off)
        _copy_idx_row(didx, dbuf, off)
        pltpu.async_copy(table_hbm.at[sidx.at[0]], rows, sem).wait()
        pltpu.sync_copy(rows, acc.at[didx.at[0]], add=True)


def _carry_tail(sbuf, dbuf, cnt):
    nb = lax.shift_right_logical(cnt, 7)
    off = nb * 128
    for k in range(128 // _L):
        sv = sbuf[pl.ds(off + k * _L, _L)]
        dv = dbuf[pl.ds(off + k * _L, _L)]
        sbuf[pl.ds(k * _L, _L)] = sv
        dbuf[pl.ds(k * _L, _L)] = dv
    return cnt - off, nb


_L = 16
_A2_ROWS = 10112
_A2_PAD_DST = 10016


def _sc_layer2_scatter(nef, edge2, src2, dst2):
    e_per = N_EDGES // _NW
    sec = 2000
    n_sec = e_per // sec
    cap = sec + 272

    @functools.partial(
        pl.kernel,
        out_type=jax.ShapeDtypeStruct((_NC, N_NODES, H), jnp.float32),
        mesh=_sc_mesh(),
        compiler_params=pltpu.CompilerParams(needs_layout_passes=False),
        scratch_types=[
            pltpu.VMEM((sec,), jnp.int32),
            pltpu.VMEM((sec,), jnp.int32),
            pltpu.VMEM((cap,), jnp.int32),
            pltpu.VMEM((cap,), jnp.int32),
            pltpu.VMEM((cap,), jnp.int32),
            pltpu.VMEM((cap,), jnp.int32),
            pltpu.VMEM((1, 128), jnp.int32),
            pltpu.VMEM((1, 128), jnp.int32),
            pltpu.VMEM((128, H), jnp.float32),
            pltpu.VMEM_SHARED((_A2_ROWS, H), jnp.float32),
            pltpu.SemaphoreType.DMA,
        ],
    )
    def layer2_k(nef_hbm, edge2_hbm, src_hbm, dst_hbm, out_hbm,
                 rsrc, rdst, asrc, adst, bsrc, bdst, sidx, didx, rows, acc,
                 sem):
        cid = lax.axis_index("c")
        sid = lax.axis_index("s")
        wid = sid * _NC + cid
        base = wid * e_per

        _zero_vmem_rows(rows, 128)
        zbase = sid * 632
        for j in range(4):
            pltpu.sync_copy(rows, acc.at[pl.ds(zbase + j * 128, 128)])
        pltpu.sync_copy(rows.at[pl.ds(0, 120)],
                        acc.at[pl.ds(zbase + 512, 120)])
        plsc.subcore_barrier()

        def scan_section(s_i, carry):
            ca, cb = carry
            off = base + s_i * sec
            pltpu.sync_copy(src_hbm.at[pl.ds(off, sec)], rsrc)
            pltpu.sync_copy(dst_hbm.at[pl.ds(off, sec)], rdst)

            def scan_body(i, carry):
                ca, cb = carry
                s = rsrc[pl.ds(i * _L, _L)]
                d = rdst[pl.ds(i * _L, _L)]
                keep = d < N_NODES
                ma = jnp.logical_and(keep, s < N_NODES)
                mb = jnp.logical_and(keep, s >= N_NODES)
                plsc.store_compressed(asrc.at[pl.ds(ca, _L)], s, mask=ma)
                plsc.store_compressed(adst.at[pl.ds(ca, _L)], d, mask=ma)
                plsc.store_compressed(bsrc.at[pl.ds(cb, _L)], s, mask=mb)
                plsc.store_compressed(bdst.at[pl.ds(cb, _L)], d, mask=mb)
                ca = ca + plsc.all_reduce_population_count(ma)[0]
                cb = cb + plsc.all_reduce_population_count(mb)[0]
                return ca, cb

            ca, cb = pl.loop(0, sec // _L, init_carry=(ca, cb),
                             unroll=2)(scan_body)
            nba = lax.shift_right_logical(ca, 7)
            _flush_batches(asrc, adst, nba, sidx, didx, rows, acc, nef_hbm,
                           sem)
            ca, _ = _carry_tail(asrc, adst, ca)
            nbb = lax.shift_right_logical(cb, 7)
            _flush_batches(bsrc, bdst, nbb, sidx, didx, rows, acc, edge2_hbm,
                           sem)
            cb, _ = _carry_tail(bsrc, bdst, cb)
            return ca, cb

        ca, cb = pl.loop(0, n_sec, init_carry=(jnp.int32(0), jnp.int32(0)))(
            scan_section)

        _pad_tail(asrc, ca, 0)
        _pad_tail(adst, ca, _A2_PAD_DST)
        _flush_batches(asrc, adst, lax.shift_right_logical(ca + 127, 7),
                       sidx, didx, rows, acc, nef_hbm, sem)
        _pad_tail(bsrc, cb, N_NODES)
        _pad_tail(bdst, cb, _A2_PAD_DST)
        _flush_batches(bsrc, bdst, lax.shift_right_logical(cb + 127, 7),
                       sidx, didx, rows, acc, edge2_hbm, sem)

        plsc.subcore_barrier()

        wbase = sid * 624
        pltpu.sync_copy(acc.at[pl.ds(wbase, 624)],
                        out_hbm.at[cid].at[pl.ds(wbase, 624)])

        @pl.when(sid == _NS - 1)
        def _():
            pltpu.sync_copy(acc.at[pl.ds(9984, 16)],
                            out_hbm.at[cid].at[pl.ds(9984, 16)])

    return layer2_k(nef, edge2, src2, dst2)


_A1_BIN = 13056
_A1_NBINS = 26
_A1_ROWS = _A1_BIN + 8
_A1_OUT = _A1_BIN * _A1_NBINS
_LOD_ZROW = N_NODES + N_EDGES


def _sc_layer1_scatter(lod, src1, dst1):
    e_per = N_EDGES // _NS
    sec = 2000
    n_sec = e_per // sec
    cap = sec + 176

    @functools.partial(
        pl.kernel,
        out_type=jax.ShapeDtypeStruct((_A1_OUT, H), jnp.float32),
        mesh=_sc_mesh(),
        compiler_params=pltpu.CompilerParams(needs_layout_passes=False),
        scratch_types=[
            pltpu.VMEM((sec,), jnp.int32),
            pltpu.VMEM((sec,), jnp.int32),
            pltpu.VMEM((cap,), jnp.int32),
            pltpu.VMEM((cap,), jnp.int32),
            pltpu.VMEM((1, 128), jnp.int32),
            pltpu.VMEM((1, 128), jnp.int32),
            pltpu.VMEM((128, H), jnp.float32),
            pltpu.VMEM_SHARED((_A1_ROWS, H), jnp.float32),
            pltpu.SemaphoreType.DMA,
        ],
    )
    def layer1_k(lod_hbm, src_hbm, dst_hbm, out_hbm,
                 rsrc, rdst, csrc, cdst, sidx, didx, rows, acc, sem):
        cid = lax.axis_index("c")
        sid = lax.axis_index("s")
        base = sid * e_per

        _zero_vmem_rows(rows, 128)

        @pl.loop(0, _A1_NBINS // _NC)
        def _(bi):
            b = bi * _NC + cid
            lo = b * _A1_BIN

            zbase = sid * 816
            for j in range(6):
                pltpu.sync_copy(rows, acc.at[pl.ds(zbase + j * 128, 128)])
            pltpu.sync_copy(rows.at[pl.ds(0, 48)],
                            acc.at[pl.ds(zbase + 768, 48)])
            plsc.subcore_barrier()

            def scan_section(s_i, cnt):
                off = base + s_i * sec
                pltpu.sync_copy(src_hbm.at[pl.ds(off, sec)], rsrc)
                pltpu.sync_copy(dst_hbm.at[pl.ds(off, sec)], rdst)

                def scan_body(i, cnt):
                    s = rsrc[pl.ds(i * _L, _L)]
                    d = rdst[pl.ds(i * _L, _L)]
                    dl = d - lo
                    m = plsc.bitcast(dl, jnp.uint32) < jnp.uint32(_A1_BIN)
                    plsc.store_compressed(csrc.at[pl.ds(cnt, _L)], s, mask=m)
                    plsc.store_compressed(cdst.at[pl.ds(cnt, _L)], dl, mask=m)
                    return cnt + plsc.all_reduce_population_count(m)[0]

                cnt = pl.loop(0, sec // _L, init_carry=cnt,
                              unroll=5)(scan_body)
                nb = lax.shift_right_logical(cnt, 7)
                _flush_batches(csrc, cdst, nb, sidx, didx, rows, acc,
                               lod_hbm, sem)
                cnt, _ = _carry_tail(csrc, cdst, cnt)
                return cnt

            cnt = pl.loop(0, n_sec, init_carry=jnp.int32(0))(scan_section)
            _pad_tail(csrc, cnt, _LOD_ZROW)
            _pad_tail(cdst, cnt, _A1_BIN)
            _flush_batches(csrc, cdst, lax.shift_right_logical(cnt + 127, 7),
                           sidx, didx, rows, acc, lod_hbm, sem)

            plsc.subcore_barrier()
            pltpu.sync_copy(acc.at[pl.ds(sid * 816, 816)],
                            out_hbm.at[pl.ds(lo + sid * 816, 816)])
            _zero_vmem_rows(rows, 128)

    return layer1_k(lod, src1, dst1)


def _lod_body(nef_ref, src_ref, dst_ref, dist_ref, w1_ref, w2_ref, w3_ref,
              b_ref, o_ref):
    i = pl.program_id(0)

    @pl.when(i < 10)
    def _():
        o_ref[...] = nef_ref[...]

    @pl.when(jnp.logical_and(i >= 10, i < 330))
    def _():
        acc = jnp.dot(src_ref[...], w1_ref[...],
                      preferred_element_type=jnp.float32)
        acc = acc + jnp.dot(dst_ref[...], w2_ref[...],
                            preferred_element_type=jnp.float32)
        acc = acc + jnp.dot(dist_ref[...], w3_ref[...],
                            preferred_element_type=jnp.float32)
        o_ref[...] = jnp.maximum(acc + b_ref[...], 0.0)

    @pl.when(i == 330)
    def _():
        o_ref[...] = jnp.zeros_like(o_ref)


def _build_lod(nef, sd_feat, dist_feat, w1, w2, w3, b):
    e_blk = N_EDGES // ROW_BLK
    grid = 331

    def eb(i):
        return jnp.clip(i - 10, 0, e_blk - 1)

    row_spec = pl.BlockSpec((ROW_BLK, H), lambda i: (eb(i), 0))
    dst_spec = pl.BlockSpec((ROW_BLK, H), lambda i: (eb(i) + e_blk, 0))
    nef_spec = pl.BlockSpec((ROW_BLK, H), lambda i: (jnp.minimum(i, 9), 0))
    w_spec = pl.BlockSpec((H, H), lambda i: (0, 0))
    b_spec = pl.BlockSpec((1, H), lambda i: (0, 0))
    return pl.pallas_call(
        _lod_body,
        grid=(grid,),
        in_specs=[nef_spec, row_spec, dst_spec, row_spec, w_spec, w_spec,
                  w_spec, b_spec],
        out_specs=pl.BlockSpec((ROW_BLK, H), lambda i: (i, 0)),
        out_shape=jax.ShapeDtypeStruct((331 * ROW_BLK, H), jnp.float32),
    )(nef, sd_feat, sd_feat, dist_feat, w1, w2, w3, b.reshape(1, H))


def _mm_relu_body(x_ref, w_ref, o_ref):
    o_ref[...] = jnp.maximum(
        jnp.dot(x_ref[...], w_ref[...],
                preferred_element_type=jnp.float32), 0.0)


def _mm_relu(x, w, n_rows=None):
    n = n_rows or x.shape[0]
    grid = n // ROW_BLK
    row_spec = pl.BlockSpec((ROW_BLK, H), lambda i: (i, 0))
    w_spec = pl.BlockSpec((H, H), lambda i: (0, 0))
    return pl.pallas_call(
        _mm_relu_body,
        grid=(grid,),
        in_specs=[row_spec, w_spec],
        out_specs=row_spec,
        out_shape=jax.ShapeDtypeStruct((n, H), jnp.float32),
    )(x, w)


def _mm2_body(buf_ref, x0_ref, x1_ref, w_ref, o_ref):
    del buf_ref
    o_ref[...] = jnp.maximum(
        jnp.dot(x0_ref[0] + x1_ref[0], w_ref[...],
                preferred_element_type=jnp.float32), 0.0)


def _mm2_relu_into(p, w, buf):
    grid = N_NODES // ROW_BLK
    buf_spec = pl.BlockSpec((8, H), lambda i: (0, 0))
    spec0 = pl.BlockSpec((1, ROW_BLK, H), lambda i: (0, i, 0))
    spec1 = pl.BlockSpec((1, ROW_BLK, H), lambda i: (1, i, 0))
    w_spec = pl.BlockSpec((H, H), lambda i: (0, 0))
    return pl.pallas_call(
        _mm2_body,
        grid=(grid,),
        in_specs=[buf_spec, spec0, spec1, w_spec],
        out_specs=pl.BlockSpec((ROW_BLK, H), lambda i: (i, 0)),
        out_shape=jax.ShapeDtypeStruct(buf.shape, jnp.float32),
        input_output_aliases={0: 0},
    )(buf, p, p, w)


def kernel(node_edge_feat, dist_feat_order, dist_feat, W_fc, b_fc, W_g, a_src_g,
           a_dst_g, W_s, W_e_s, a_src_s, a_dst_s, a_edge_s, srcs, dsts, nids,
           eids, e2n_edge_index, e2e_edge_index, nlod, elod):
    w1, w2, w3 = W_fc[:H], W_fc[H:2 * H], W_fc[2 * H:]
    node_feat = node_edge_feat[:N_NODES]

    del node_feat
    sd_feat = _sc_gather(node_edge_feat, jnp.concatenate([srcs, dsts]))
    lod = _build_lod(node_edge_feat, sd_feat, dist_feat, w1, w2, w3, b_fc)

    a1 = _sc_layer1_scatter(lod, e2e_edge_index[0], e2e_edge_index[1])
    ne = _mm_relu(a1, W_g, n_rows=N_NODES + N_EDGES)
    out_buf = _sc_gather(ne, eids, out_rows=N_NODES + N_EDGES,
                         row_offset=N_NODES)

    p2 = _sc_layer2_scatter(node_edge_feat, out_buf,
                            e2n_edge_index[0], e2n_edge_index[1])
    return _mm2_relu_into(p2, W_s, out_buf)


# speedup vs baseline: 1.0447x; 1.0447x over previous
import functools

import jax
import jax.numpy as jnp
from jax import lax
from jax.experimental import pallas as pl
from jax.experimental.pallas import tpu as pltpu
from jax.experimental.pallas import tpu_sc as plsc

N_NODES = 10000
N_EDGES = 320000
H = 128
ROW_BLK = 1000

_NC = 2
_NS = 16
_NW = _NC * _NS
_GC = 80


def _sc_mesh():
    return plsc.VectorSubcoreMesh(
        core_axis_name="c", subcore_axis_name="s",
        num_cores=_NC, num_subcores=_NS)


def _sc_gather(table, idx, out_rows=None, row_offset=0):
    b = idx.shape[0]
    per_w = b // _NW
    assert per_w * _NW == b and per_w % _GC == 0
    n_chunks = per_w // _GC

    @functools.partial(
        pl.kernel,
        out_type=jax.ShapeDtypeStruct((out_rows or b, H), jnp.float32),
        mesh=_sc_mesh(),
        scratch_types=[
            pltpu.VMEM((2, _GC), jnp.int32),
            pltpu.VMEM((2, _GC, H), jnp.float32),
            pltpu.SemaphoreType.DMA,
            pltpu.SemaphoreType.DMA,
        ],
    )
    def gather_k(table_hbm, idx_hbm, out_hbm, idx_v, rows_v, sem0, sem1):
        wid = lax.axis_index("s") * _NC + lax.axis_index("c")
        base = wid * per_w
        sems = (sem0, sem1)

        def body(j, p):
            q = 1 - p

            @pl.when(j + 1 < n_chunks)
            def _():
                off = base + (j + 1) * _GC
                pltpu.sync_copy(idx_hbm.at[pl.ds(off, _GC)], idx_v.at[q])
                pltpu.async_copy(table_hbm.at[idx_v.at[q]], rows_v.at[q], sems[q])

            pltpu.make_async_copy(
                table_hbm.at[idx_v.at[p]], rows_v.at[p], sems[p]).wait()
            pltpu.sync_copy(
                rows_v.at[p],
                out_hbm.at[pl.ds(row_offset + base + j * _GC, _GC)])

        pltpu.sync_copy(idx_hbm.at[pl.ds(base, _GC)], idx_v.at[0])
        pltpu.async_copy(table_hbm.at[idx_v.at[0]], rows_v.at[0], sem0)

        @pl.loop(0, 2 * (n_chunks // 2), step=2)
        def _(i):
            body(i, 0)
            body(i + 1, 1)

        if n_chunks % 2:
            body(n_chunks - 1, 0)

    return gather_k(table, idx)


def _zero_vmem_rows(buf, nrows):
    z = jnp.zeros((_L,), jnp.float32)

    @pl.loop(0, nrows)
    def _(i):
        for k in range(H // _L):
            buf[i, pl.ds(k * _L, _L)] = z


def _copy_idx_row(dst2d, src1d, off):
    for k in range(128 // _L):
        dst2d[0, pl.ds(k * _L, _L)] = src1d[pl.ds(off + k * _L, _L)]


def _pad_tail(buf, cnt, value):
    v = jnp.full((_L,), value, jnp.int32) + lax.rem(
        lax.iota(jnp.int32, _L), jnp.int32(8))
    for k in range(128 // _L):
        buf[pl.ds(cnt + k * _L, _L)] = v


def _flush_batches(sbuf, dbuf, nb, sidx, didx, rows, acc, table_hbm, sem):
    @pl.loop(0, nb)
    def _(k):
        off = k * 128
        _copy_idx_row(sidx, sbuf, off)
        _copy_idx_row(didx, dbuf, off)
        pltpu.async_copy(table_hbm.at[sidx.at[0]], rows, sem).wait()
        pltpu.sync_copy(rows, acc.at[didx.at[0]], add=True)


def _carry_tail(sbuf, dbuf, cnt):
    nb = lax.shift_right_logical(cnt, 7)
    off = nb * 128
    for k in range(128 // _L):
        sv = sbuf[pl.ds(off + k * _L, _L)]
        dv = dbuf[pl.ds(off + k * _L, _L)]
        sbuf[pl.ds(k * _L, _L)] = sv
        dbuf[pl.ds(k * _L, _L)] = dv
    return cnt - off, nb


_L = 16
_A2_ROWS = 10112
_A2_PAD_DST = 10016


def _sc_layer2_scatter(nef, edge2, src2, dst2):
    e_per = N_EDGES // _NW
    sec = 2000
    n_sec = e_per // sec
    cap = sec + 272

    @functools.partial(
        pl.kernel,
        out_type=jax.ShapeDtypeStruct((_NC, N_NODES, H), jnp.float32),
        mesh=_sc_mesh(),
        compiler_params=pltpu.CompilerParams(needs_layout_passes=False),
        scratch_types=[
            pltpu.VMEM((sec,), jnp.int32),
            pltpu.VMEM((sec,), jnp.int32),
            pltpu.VMEM((cap,), jnp.int32),
            pltpu.VMEM((cap,), jnp.int32),
            pltpu.VMEM((cap,), jnp.int32),
            pltpu.VMEM((cap,), jnp.int32),
            pltpu.VMEM((1, 128), jnp.int32),
            pltpu.VMEM((1, 128), jnp.int32),
            pltpu.VMEM((128, H), jnp.float32),
            pltpu.VMEM_SHARED((_A2_ROWS, H), jnp.float32),
            pltpu.SemaphoreType.DMA,
        ],
    )
    def layer2_k(nef_hbm, edge2_hbm, src_hbm, dst_hbm, out_hbm,
                 rsrc, rdst, asrc, adst, bsrc, bdst, sidx, didx, rows, acc,
                 sem):
        cid = lax.axis_index("c")
        sid = lax.axis_index("s")
        wid = sid * _NC + cid
        base = wid * e_per

        _zero_vmem_rows(rows, 128)
        zbase = sid * 632
        for j in range(4):
            pltpu.sync_copy(rows, acc.at[pl.ds(zbase + j * 128, 128)])
        pltpu.sync_copy(rows.at[pl.ds(0, 120)],
                        acc.at[pl.ds(zbase + 512, 120)])
        plsc.subcore_barrier()

        def scan_section(s_i, carry):
            ca, cb = carry
            off = base + s_i * sec
            pltpu.sync_copy(src_hbm.at[pl.ds(off, sec)], rsrc)
            pltpu.sync_copy(dst_hbm.at[pl.ds(off, sec)], rdst)

            def scan_body(i, carry):
                ca, cb = carry
                s = rsrc[pl.ds(i * _L, _L)]
                d = rdst[pl.ds(i * _L, _L)]
                keep = d < N_NODES
                ma = jnp.logical_and(keep, s < N_NODES)
                mb = jnp.logical_and(keep, s >= N_NODES)
                plsc.store_compressed(asrc.at[pl.ds(ca, _L)], s, mask=ma)
                plsc.store_compressed(adst.at[pl.ds(ca, _L)], d, mask=ma)
                plsc.store_compressed(bsrc.at[pl.ds(cb, _L)], s, mask=mb)
                plsc.store_compressed(bdst.at[pl.ds(cb, _L)], d, mask=mb)
                ca = ca + plsc.all_reduce_population_count(ma)[0]
                cb = cb + plsc.all_reduce_population_count(mb)[0]
                return ca, cb

            ca, cb = pl.loop(0, sec // _L, init_carry=(ca, cb),
                             unroll=2)(scan_body)
            nba = lax.shift_right_logical(ca, 7)
            _flush_batches(asrc, adst, nba, sidx, didx, rows, acc, nef_hbm,
                           sem)
            ca, _ = _carry_tail(asrc, adst, ca)
            nbb = lax.shift_right_logical(cb, 7)
            _flush_batches(bsrc, bdst, nbb, sidx, didx, rows, acc, edge2_hbm,
                           sem)
            cb, _ = _carry_tail(bsrc, bdst, cb)
            return ca, cb

        ca, cb = pl.loop(0, n_sec, init_carry=(jnp.int32(0), jnp.int32(0)))(
            scan_section)

        _pad_tail(asrc, ca, 0)
        _pad_tail(adst, ca, _A2_PAD_DST)
        _flush_batches(asrc, adst, lax.shift_right_logical(ca + 127, 7),
                       sidx, didx, rows, acc, nef_hbm, sem)
        _pad_tail(bsrc, cb, N_NODES)
        _pad_tail(bdst, cb, _A2_PAD_DST)
        _flush_batches(bsrc, bdst, lax.shift_right_logical(cb + 127, 7),
                       sidx, didx, rows, acc, edge2_hbm, sem)

        plsc.subcore_barrier()

        wbase = sid * 624
        pltpu.sync_copy(acc.at[pl.ds(wbase, 624)],
                        out_hbm.at[cid].at[pl.ds(wbase, 624)])

        @pl.when(sid == _NS - 1)
        def _():
            pltpu.sync_copy(acc.at[pl.ds(9984, 16)],
                            out_hbm.at[cid].at[pl.ds(9984, 16)])

    return layer2_k(nef, edge2, src2, dst2)


_A1_BIN = 12032
_A1_NBINS = 28
_A1_ROWS = _A1_BIN + 8
_A1_OUT = _A1_BIN * _A1_NBINS
_LOD_ZROW = N_NODES + N_EDGES


def _sc_layer1_scatter(lod, src1, dst1):
    e_per = N_EDGES // _NS
    sec = 4000
    n_sec = e_per // sec
    cap = sec + 272

    @functools.partial(
        pl.kernel,
        out_type=jax.ShapeDtypeStruct((_A1_OUT, H), jnp.float32),
        mesh=_sc_mesh(),
        compiler_params=pltpu.CompilerParams(needs_layout_passes=False),
        scratch_types=[
            pltpu.VMEM((sec,), jnp.int32),
            pltpu.VMEM((sec,), jnp.int32),
            pltpu.VMEM((cap,), jnp.int32),
            pltpu.VMEM((cap,), jnp.int32),
            pltpu.VMEM((1, 128), jnp.int32),
            pltpu.VMEM((1, 128), jnp.int32),
            pltpu.VMEM((128, H), jnp.float32),
            pltpu.VMEM_SHARED((_A1_ROWS, H), jnp.float32),
            pltpu.SemaphoreType.DMA,
        ],
    )
    def layer1_k(lod_hbm, src_hbm, dst_hbm, out_hbm,
                 rsrc, rdst, csrc, cdst, sidx, didx, rows, acc, sem):
        cid = lax.axis_index("c")
        sid = lax.axis_index("s")
        base = sid * e_per

        _zero_vmem_rows(rows, 128)

        @pl.loop(0, _A1_NBINS // _NC)
        def _(bi):
            b = bi * _NC + cid
            lo = b * _A1_BIN

            zbase = sid * 752
            for j in range(5):
                pltpu.sync_copy(rows, acc.at[pl.ds(zbase + j * 128, 128)])
            pltpu.sync_copy(rows.at[pl.ds(0, 112)],
                            acc.at[pl.ds(zbase + 640, 112)])
            plsc.subcore_barrier()

            def scan_section(s_i, cnt):
                off = base + s_i * sec
                pltpu.sync_copy(src_hbm.at[pl.ds(off, sec)], rsrc)
                pltpu.sync_copy(dst_hbm.at[pl.ds(off, sec)], rdst)

                def scan_body(i, cnt):
                    s = rsrc[pl.ds(i * _L, _L)]
                    d = rdst[pl.ds(i * _L, _L)]
                    dl = d - lo
                    m = plsc.bitcast(dl, jnp.uint32) < jnp.uint32(_A1_BIN)
                    plsc.store_compressed(csrc.at[pl.ds(cnt, _L)], s, mask=m)
                    plsc.store_compressed(cdst.at[pl.ds(cnt, _L)], dl, mask=m)
                    return cnt + plsc.all_reduce_population_count(m)[0]

                cnt = pl.loop(0, sec // _L, init_carry=cnt,
                              unroll=5)(scan_body)
                nb = lax.shift_right_logical(cnt, 7)
                _flush_batches(csrc, cdst, nb, sidx, didx, rows, acc,
                               lod_hbm, sem)
                cnt, _ = _carry_tail(csrc, cdst, cnt)
                return cnt

            cnt = pl.loop(0, n_sec, init_carry=jnp.int32(0))(scan_section)
            _pad_tail(csrc, cnt, _LOD_ZROW)
            _pad_tail(cdst, cnt, _A1_BIN)
            _flush_batches(csrc, cdst, lax.shift_right_logical(cnt + 127, 7),
                           sidx, didx, rows, acc, lod_hbm, sem)

            plsc.subcore_barrier()
            pltpu.sync_copy(acc.at[pl.ds(sid * 752, 752)],
                            out_hbm.at[pl.ds(lo + sid * 752, 752)])
            _zero_vmem_rows(rows, 128)

    return layer1_k(lod, src1, dst1)


def _lod_body(nef_ref, src_ref, dst_ref, dist_ref, w1_ref, w2_ref, w3_ref,
              b_ref, o_ref):
    i = pl.program_id(0)

    @pl.when(i < 10)
    def _():
        o_ref[...] = nef_ref[...]

    @pl.when(jnp.logical_and(i >= 10, i < 330))
    def _():
        acc = jnp.dot(src_ref[...], w1_ref[...],
                      preferred_element_type=jnp.float32)
        acc = acc + jnp.dot(dst_ref[...], w2_ref[...],
                            preferred_element_type=jnp.float32)
        acc = acc + jnp.dot(dist_ref[...], w3_ref[...],
                            preferred_element_type=jnp.float32)
        o_ref[...] = jnp.maximum(acc + b_ref[...], 0.0)

    @pl.when(i == 330)
    def _():
        o_ref[...] = jnp.zeros_like(o_ref)


def _build_lod(nef, sd_feat, dist_feat, w1, w2, w3, b):
    e_blk = N_EDGES // ROW_BLK
    grid = 331

    def eb(i):
        return jnp.clip(i - 10, 0, e_blk - 1)

    row_spec = pl.BlockSpec((ROW_BLK, H), lambda i: (eb(i), 0))
    dst_spec = pl.BlockSpec((ROW_BLK, H), lambda i: (eb(i) + e_blk, 0))
    nef_spec = pl.BlockSpec((ROW_BLK, H), lambda i: (jnp.minimum(i, 9), 0))
    w_spec = pl.BlockSpec((H, H), lambda i: (0, 0))
    b_spec = pl.BlockSpec((1, H), lambda i: (0, 0))
    return pl.pallas_call(
        _lod_body,
        grid=(grid,),
        in_specs=[nef_spec, row_spec, dst_spec, row_spec, w_spec, w_spec,
                  w_spec, b_spec],
        out_specs=pl.BlockSpec((ROW_BLK, H), lambda i: (i, 0)),
        out_shape=jax.ShapeDtypeStruct((331 * ROW_BLK, H), jnp.float32),
    )(nef, sd_feat, sd_feat, dist_feat, w1, w2, w3, b.reshape(1, H))


def _mm_relu_body(x_ref, w_ref, o_ref):
    o_ref[...] = jnp.maximum(
        jnp.dot(x_ref[...], w_ref[...],
                preferred_element_type=jnp.float32), 0.0)


def _mm_relu(x, w, n_rows=None):
    n = n_rows or x.shape[0]
    grid = n // ROW_BLK
    row_spec = pl.BlockSpec((ROW_BLK, H), lambda i: (i, 0))
    w_spec = pl.BlockSpec((H, H), lambda i: (0, 0))
    return pl.pallas_call(
        _mm_relu_body,
        grid=(grid,),
        in_specs=[row_spec, w_spec],
        out_specs=row_spec,
        out_shape=jax.ShapeDtypeStruct((n, H), jnp.float32),
    )(x, w)


def _mm2_body(buf_ref, x0_ref, x1_ref, w_ref, o_ref):
    del buf_ref
    o_ref[...] = jnp.maximum(
        jnp.dot(x0_ref[0] + x1_ref[0], w_ref[...],
                preferred_element_type=jnp.float32), 0.0)


def _mm2_relu_into(p, w, buf):
    grid = N_NODES // ROW_BLK
    buf_spec = pl.BlockSpec((8, H), lambda i: (0, 0))
    spec0 = pl.BlockSpec((1, ROW_BLK, H), lambda i: (0, i, 0))
    spec1 = pl.BlockSpec((1, ROW_BLK, H), lambda i: (1, i, 0))
    w_spec = pl.BlockSpec((H, H), lambda i: (0, 0))
    return pl.pallas_call(
        _mm2_body,
        grid=(grid,),
        in_specs=[buf_spec, spec0, spec1, w_spec],
        out_specs=pl.BlockSpec((ROW_BLK, H), lambda i: (i, 0)),
        out_shape=jax.ShapeDtypeStruct(buf.shape, jnp.float32),
        input_output_aliases={0: 0},
    )(buf, p, p, w)


def kernel(node_edge_feat, dist_feat_order, dist_feat, W_fc, b_fc, W_g, a_src_g,
           a_dst_g, W_s, W_e_s, a_src_s, a_dst_s, a_edge_s, srcs, dsts, nids,
           eids, e2n_edge_index, e2e_edge_index, nlod, elod):
    w1, w2, w3 = W_fc[:H], W_fc[H:2 * H], W_fc[2 * H:]
    node_feat = node_edge_feat[:N_NODES]

    del node_feat
    sd_feat = _sc_gather(node_edge_feat, jnp.concatenate([srcs, dsts]))
    lod = _build_lod(node_edge_feat, sd_feat, dist_feat, w1, w2, w3, b_fc)

    a1 = _sc_layer1_scatter(lod, e2e_edge_index[0], e2e_edge_index[1])
    ne = _mm_relu(a1, W_g, n_rows=N_NODES + N_EDGES)
    out_buf = _sc_gather(ne, eids, out_rows=N_NODES + N_EDGES,
                         row_offset=N_NODES)

    p2 = _sc_layer2_scatter(node_edge_feat, out_buf,
                            e2n_edge_index[0], e2n_edge_index[1])
    return _mm2_relu_into(p2, W_s, out_buf)
